# baseline (speedup 1.0000x reference)
import functools

import jax
import jax.numpy as jnp
from jax import lax
from jax.experimental import pallas as pl
from jax.experimental.pallas import tpu as pltpu
from jax.experimental.pallas import tpu_sc as plsc

NUM_ROWS = 100001
E = 1024
B = 16384
L = 16

NC = 2
NS = 16
NW = NC * NS
RPW = B // NW
FPW = 2 * RPW
CH = 8
NBUF = 8
NCHUNK = FPW // CH
BR_PER_CH = CH // 2
G = 8
NJ = E // L
JU = 1
NSEG = 4


def _sc_body(idx_hbm, table_hbm, wmu_hbm, wls_hbm, weps_hbm, bp_hbm,
             out_hbm,
             idx_v, rows_v, w_v, t0_v, t1_v, t2_v, bp_v, red_v, out_v,
             *sems):
    wid = lax.axis_index("s") * NC + lax.axis_index("c")

    pltpu.sync_copy(idx_hbm.at[pl.ds(wid * FPW, FPW)], idx_v)

    pltpu.sync_copy(wmu_hbm, t0_v)
    pltpu.sync_copy(wls_hbm, t1_v)
    pltpu.sync_copy(weps_hbm, t2_v)

    def w_body(j, _):
        o = j * 2 * L
        for r in range(2):
            wa = (t0_v[r, pl.ds(o, L)]
                  + jnp.exp(t1_v[r, pl.ds(o, L)]) * t2_v[r, pl.ds(o, L)])
            wb = (t0_v[r, pl.ds(o + L, L)]
                  + jnp.exp(t1_v[r, pl.ds(o + L, L)]) * t2_v[r, pl.ds(o + L, L)])
            packed = plsc.pack(wa, wb, format=plsc.PackFormat.INTERLEAVED)
            w_v[pl.ds((r * 2 * E + o) // 2, L)] = plsc.bitcast(
                packed, jnp.int32)
        return 0

    lax.fori_loop(0, E // L, w_body, 0)

    pltpu.sync_copy(bp_hbm, bp_v)
    b_vec = bp_v[0, :] + jnp.exp(bp_v[1, :]) * bp_v[2, :]

    lane = jax.lax.iota(jnp.int32, L)
    col_base = lane * L

    def gather(chunk_i, buf):
        return pltpu.make_async_copy(
            table_hbm.at[idx_v.at[pl.ds(chunk_i * CH, CH)]],
            rows_v.at[buf], sems[buf])

    for b in range(NBUF):
        gather(b, b).start()

    def compute_pair(chunk_i, bufbase):
        fzero = jnp.zeros((L,), jnp.float32)
        faccs = [fzero] * (2 * G)
        fzero = jnp.zeros((L,), jnp.float32)
        faccs = [fzero] * (2 * G)
        for seg in range(NSEG):
            zero = jnp.zeros((2 * L,), jnp.bfloat16)
            accs = tuple(zero for _ in range(2 * G))
            seg_lo = seg * (NJ // 2 // NSEG)
            seg_hi = seg_lo + NJ // 2 // NSEG

            @plsc.parallel_loop(seg_lo, seg_hi, unroll=JU, carry=accs)
            def j_loop(j, accs):
                o = j * 2 * L
                bf = jnp.bfloat16
                w0a = plsc.bitcast(w_v[pl.ds(o // 2, L)], bf)
                w1a = plsc.bitcast(w_v[pl.ds((2 * E + o) // 2, L)], bf)
                w0b = plsc.bitcast(w_v[pl.ds((E + o) // 2, L)], bf)
                w1b = plsc.bitcast(w_v[pl.ds((3 * E + o) // 2, L)], bf)
                new = []
                for g in range(G):
                    fr0 = 2 * g
                    fr1 = 2 * g + 1
                    rb0 = plsc.pack(
                        rows_v[bufbase + fr0 // CH, fr0 % CH, pl.ds(o, L)],
                        rows_v[bufbase + fr0 // CH, fr0 % CH, pl.ds(o + L, L)],
                        format=plsc.PackFormat.INTERLEAVED)
                    rb1 = plsc.pack(
                        rows_v[bufbase + fr1 // CH, fr1 % CH, pl.ds(o, L)],
                        rows_v[bufbase + fr1 // CH, fr1 % CH, pl.ds(o + L, L)],
                        format=plsc.PackFormat.INTERLEAVED)
                    new.append(accs[2 * g] + rb0 * w0a + rb1 * w0b)
                    new.append(accs[2 * g + 1] + rb0 * w1a + rb1 * w1b)
                return tuple(new)

            for k in range(2 * G):
                ua, ub = plsc.unpack(j_loop[k],
                                     format=plsc.PackFormat.INTERLEAVED)
                faccs[k] = faccs[k] + ua + ub

        for k in range(2 * G):
            red_v[pl.ds(L * k, L)] = faccs[k]
        tot = b_vec
        for l in range(L):
            tot = tot + plsc.load_gather(red_v, [col_base + l])
        out_v[pl.ds(CH * chunk_i, L)] = tot

    def c_body(c, _):
        for bufbase in range(0, NBUF, 2):
            chunk_i = NBUF * c + bufbase
            gather(chunk_i, bufbase).wait()
            gather(chunk_i + 1, bufbase + 1).wait()
            compute_pair(chunk_i, bufbase)
            gather(chunk_i + NBUF, bufbase).start()
            gather(chunk_i + 1 + NBUF, bufbase + 1).start()
        return 0

    lax.fori_loop(0, NCHUNK // NBUF - 1, c_body, 0)
    for bufbase in range(0, NBUF, 2):
        chunk_i = NCHUNK - NBUF + bufbase
        gather(chunk_i, bufbase).wait()
        gather(chunk_i + 1, bufbase + 1).wait()
        compute_pair(chunk_i, bufbase)

    pltpu.sync_copy(out_v, out_hbm.at[pl.ds(wid * 2 * RPW, 2 * RPW)])


@jax.jit
def _sc_call(idx2d, table, wmu, wls, weps, bpack):
    mesh = plsc.VectorSubcoreMesh(core_axis_name="c", subcore_axis_name="s")
    f = functools.partial(
        pl.kernel,
        mesh=mesh,
        compiler_params=pltpu.CompilerParams(
            needs_layout_passes=False,
            disable_bounds_checks=True,
            disable_semaphore_checks=True,
            skip_device_barrier=True,
        ),
        out_type=jax.ShapeDtypeStruct((2 * B,), jnp.float32),
        scratch_types=[
            pltpu.VMEM((FPW,), jnp.int32),
            pltpu.VMEM((NBUF, CH, E), jnp.float32),
            pltpu.VMEM((2 * E,), jnp.int32),
            pltpu.VMEM((2, 2 * E), jnp.float32),
            pltpu.VMEM((2, 2 * E), jnp.float32),
            pltpu.VMEM((2, 2 * E), jnp.float32),
            pltpu.VMEM((3, L), jnp.float32),
            pltpu.VMEM((2 * G * L,), jnp.float32),
            pltpu.VMEM((2 * RPW,), jnp.float32),
        ] + [pltpu.SemaphoreType.DMA] * NBUF,
    )(_sc_body)
    return f(idx2d, table, wmu, wls, weps, bpack)


def kernel(x, table, weight_mu, weight_log_sigma, bias_mu, bias_log_sigma,
           eps_w, eps_b):
    idx2d = x.astype(jnp.int32).reshape(2 * B)
    bpack = jnp.stack([
        jnp.tile(bias_mu, L // 2),
        jnp.tile(bias_log_sigma, L // 2),
        jnp.tile(eps_b, L // 2),
    ]).astype(jnp.float32)
    out = _sc_call(idx2d, table, weight_mu, weight_log_sigma, eps_w, bpack)
    return out.reshape(B, 2)


# speedup vs baseline: 1.1907x; 1.1907x over previous
import functools

import jax
import jax.numpy as jnp
from jax import lax
from jax.experimental import pallas as pl
from jax.experimental.pallas import tpu as pltpu
from jax.experimental.pallas import tpu_sc as plsc

NUM_ROWS = 100001
E = 1024
B = 16384
L = 16

NC = 2
NS = 16
NW = NC * NS
RPW = B // NW
FPW = 2 * RPW
CH = 8
NBUF = 8
NCHUNK = FPW // CH
BR_PER_CH = CH // 2
G = 8
NJ = E // L
JU = 1


def _sc_body(idx_hbm, table_hbm, wmu_hbm, wls_hbm, weps_hbm, bp_hbm,
             out_hbm,
             idx_v, rows_v, w_v, t0_v, t1_v, t2_v, bp_v, red_v, out_v,
             *sems):
    wid = lax.axis_index("s") * NC + lax.axis_index("c")

    pltpu.sync_copy(idx_hbm.at[pl.ds(wid * FPW, FPW)], idx_v)

    pltpu.sync_copy(wmu_hbm, t0_v)
    pltpu.sync_copy(wls_hbm, t1_v)
    pltpu.sync_copy(weps_hbm, t2_v)

    def w_body(j, _):
        o = j * L
        for r in range(2):
            w_v[r, pl.ds(o, L)] = (
                t0_v[r, pl.ds(o, L)]
                + jnp.exp(t1_v[r, pl.ds(o, L)]) * t2_v[r, pl.ds(o, L)])
        return 0

    lax.fori_loop(0, 2 * E // L, w_body, 0)

    pltpu.sync_copy(bp_hbm, bp_v)
    b_vec = bp_v[0, :] + jnp.exp(bp_v[1, :]) * bp_v[2, :]

    lane = jax.lax.iota(jnp.int32, L)
    col_base = lane * L

    def gather(chunk_i, buf):
        return pltpu.make_async_copy(
            table_hbm.at[idx_v.at[pl.ds(chunk_i * CH, CH)]],
            rows_v.at[buf], sems[buf])

    for b in range(NBUF):
        gather(b, b).start()

    def compute_pair(chunk_i, bufbase):
        zero = jnp.zeros((L,), jnp.float32)
        accs = tuple(zero for _ in range(2 * G))

        @plsc.parallel_loop(0, NJ, unroll=JU, carry=accs)
        def j_loop(j, accs):
            o = j * L
            w0a = w_v[0, pl.ds(o, L)]
            w1a = w_v[1, pl.ds(o, L)]
            w0b = w_v[0, pl.ds(E + o, L)]
            w1b = w_v[1, pl.ds(E + o, L)]
            new = []
            for g in range(G):
                fr0 = 2 * g
                fr1 = 2 * g + 1
                r0 = rows_v[bufbase + fr0 // CH, fr0 % CH, pl.ds(o, L)]
                r1 = rows_v[bufbase + fr1 // CH, fr1 % CH, pl.ds(o, L)]
                new.append(accs[2 * g] + r0 * w0a + r1 * w0b)
                new.append(accs[2 * g + 1] + r0 * w1a + r1 * w1b)
            return tuple(new)

        accs = j_loop
        for k in range(2 * G):
            red_v[pl.ds(L * k, L)] = accs[k]
        tot = b_vec
        for l in range(L):
            tot = tot + plsc.load_gather(red_v, [col_base + l])
        out_v[pl.ds(CH * chunk_i, L)] = tot

    def c_body(c, _):
        for bufbase in range(0, NBUF, 2):
            chunk_i = NBUF * c + bufbase
            gather(chunk_i, bufbase).wait()
            gather(chunk_i + 1, bufbase + 1).wait()
            compute_pair(chunk_i, bufbase)
            gather(chunk_i + NBUF, bufbase).start()
            gather(chunk_i + 1 + NBUF, bufbase + 1).start()
        return 0

    lax.fori_loop(0, NCHUNK // NBUF - 1, c_body, 0)
    for bufbase in range(0, NBUF, 2):
        chunk_i = NCHUNK - NBUF + bufbase
        gather(chunk_i, bufbase).wait()
        gather(chunk_i + 1, bufbase + 1).wait()
        compute_pair(chunk_i, bufbase)

    pltpu.sync_copy(out_v, out_hbm.at[pl.ds(wid * 2 * RPW, 2 * RPW)])


@jax.jit
def _sc_call(idx2d, table, wmu, wls, weps, bpack):
    mesh = plsc.VectorSubcoreMesh(core_axis_name="c", subcore_axis_name="s")
    f = functools.partial(
        pl.kernel,
        mesh=mesh,
        compiler_params=pltpu.CompilerParams(
            needs_layout_passes=False,
            disable_bounds_checks=True,
            disable_semaphore_checks=True,
            skip_device_barrier=True,
        ),
        out_type=jax.ShapeDtypeStruct((2 * B,), jnp.float32),
        scratch_types=[
            pltpu.VMEM((FPW,), jnp.int32),
            pltpu.VMEM((NBUF, CH, E), jnp.float32),
            pltpu.VMEM((2, 2 * E), jnp.float32),
            pltpu.VMEM((2, 2 * E), jnp.float32),
            pltpu.VMEM((2, 2 * E), jnp.float32),
            pltpu.VMEM((2, 2 * E), jnp.float32),
            pltpu.VMEM((3, L), jnp.float32),
            pltpu.VMEM((2 * G * L,), jnp.float32),
            pltpu.VMEM((2 * RPW,), jnp.float32),
        ] + [pltpu.SemaphoreType.DMA] * NBUF,
    )(_sc_body)
    return f(idx2d, table, wmu, wls, weps, bpack)


def kernel(x, table, weight_mu, weight_log_sigma, bias_mu, bias_log_sigma,
           eps_w, eps_b):
    idx2d = x.astype(jnp.int32).reshape(2 * B)
    bpack = jnp.stack([
        jnp.tile(bias_mu, L // 2),
        jnp.tile(bias_log_sigma, L // 2),
        jnp.tile(eps_b, L // 2),
    ]).astype(jnp.float32)
    out = _sc_call(idx2d, table, weight_mu, weight_log_sigma, eps_w, bpack)
    return out.reshape(B, 2)
